# Initial kernel scaffold; baseline (speedup 1.0000x reference)
import jax
import jax.numpy as jnp
from jax.experimental import pallas as pl


def kernel(inputs, hash_table, W_tiny, b_tiny, Wp0, Wp1, Wp2):
    raise NotImplementedError("write your pallas kernel here")


# speedup vs baseline: 17.3557x; 17.3557x over previous
import functools

import jax
import jax.numpy as jnp
import numpy as np
from jax import lax
from jax.experimental import pallas as pl
from jax.experimental.pallas import tpu as pltpu
from jax.experimental.pallas import tpu_sc as plsc

N_LEVELS = 16
F_PER_LEVEL = 2
T = 1 << 19
BASE_RES = 16.0
PLS = 1.5
PRIME1 = np.int32(np.uint32(2654435761).view(np.int32))

NW = 32
C = 128
NCORN = 4 * N_LEVELS


def _sc_hashgrid(x_hbm, y_hbm, table_hbm, n):
    chunks_per_w = n // (NW * C)
    mesh = plsc.VectorSubcoreMesh(core_axis_name="c", subcore_axis_name="s",
                                  num_cores=2, num_subcores=16)

    @functools.partial(
        pl.kernel,
        out_type=jax.ShapeDtypeStruct((n, 2 * N_LEVELS), jnp.float32),
        mesh=mesh,
        scratch_types=[
            pltpu.VMEM((C,), jnp.float32),
            pltpu.VMEM((C,), jnp.float32),
            pltpu.VMEM((N_LEVELS, C), jnp.float32),
            pltpu.VMEM((N_LEVELS, C), jnp.float32),
            pltpu.VMEM((NCORN * C,), jnp.int32),
            pltpu.VMEM((NCORN, C, 2), jnp.float32),
            pltpu.VMEM((C, 2 * N_LEVELS), jnp.float32),
            pltpu.SemaphoreType.DMA,
        ],
        compiler_params=pltpu.CompilerParams(use_tc_tiling_on_sc=False,
                                             needs_layout_passes=False),
    )
    def k(x_ref, y_ref, table_ref, out_ref, xv, yv, wxb, wyb, idxb, gb, fb, sem):
        wid = lax.axis_index("s") * 2 + lax.axis_index("c")
        iota = lax.iota(jnp.int32, 16)
        half = lax.shift_right_logical(iota, 1)
        parity = iota & 1

        def chunk_body(ci, _):
            base = (wid * chunks_per_w + ci) * C
            pltpu.sync_copy(x_ref.at[pl.ds(base, C)], xv)
            pltpu.sync_copy(y_ref.at[pl.ds(base, C)], yv)

            def lvl_fire(l, res):
                lbase = l * T
                for g in range(C // 16):
                    sl = pl.ds(g * 16, 16)
                    xh = xv[sl] / 30.0 + 0.5
                    yh = yv[sl] / 30.0 + 0.5
                    px = xh * res
                    py = yh * res
                    ix = px.astype(jnp.int32)
                    iy = py.astype(jnp.int32)
                    wxb[l, sl] = px - ix.astype(jnp.float32)
                    wyb[l, sl] = py - iy.astype(jnp.float32)
                    ix1 = ix + 1
                    hy0 = iy * PRIME1
                    hy1 = (iy + 1) * PRIME1
                    m = jnp.int32(T - 1)
                    idxb[pl.ds((4 * l + 0) * C + g * 16, 16)] = ((ix ^ hy0) & m) + lbase
                    idxb[pl.ds((4 * l + 1) * C + g * 16, 16)] = ((ix ^ hy1) & m) + lbase
                    idxb[pl.ds((4 * l + 2) * C + g * 16, 16)] = ((ix1 ^ hy0) & m) + lbase
                    idxb[pl.ds((4 * l + 3) * C + g * 16, 16)] = ((ix1 ^ hy1) & m) + lbase
                for c in range(4):
                    j = 4 * l + c
                    pltpu.async_copy(
                        table_ref.at[idxb.at[pl.ds(j * C, C)]], gb.at[j], sem)
                return res * 1.5

            lax.fori_loop(0, N_LEVELS, lvl_fire, jnp.float32(BASE_RES))

            def lvl_blend(l, _):
                for c in range(4):
                    j = 4 * l + c
                    pltpu.make_async_copy(
                        table_ref.at[idxb.at[pl.ds(j * C, C)]], gb.at[j], sem
                    ).wait()
                l0 = jnp.full((16,), 4 * l, jnp.int32)
                col = parity + 2 * l
                for g8 in range(C // 8):
                    pidx = half + (8 * g8)
                    wxi = plsc.load_gather(wxb, [jnp.full((16,), l, jnp.int32), pidx])
                    wyi = plsc.load_gather(wyb, [jnp.full((16,), l, jnp.int32), pidx])
                    omx = 1.0 - wxi
                    omy = 1.0 - wyi
                    g00 = plsc.load_gather(gb, [l0, pidx, parity])
                    g01 = plsc.load_gather(gb, [l0 + 1, pidx, parity])
                    g10 = plsc.load_gather(gb, [l0 + 2, pidx, parity])
                    g11 = plsc.load_gather(gb, [l0 + 3, pidx, parity])
                    acc = (g00 * (omx * omy) + g01 * (omx * wyi)
                           + g10 * (wxi * omy) + g11 * (wxi * wyi))
                    plsc.store_scatter(fb, [pidx, col], acc)
                return 0

            lax.fori_loop(0, N_LEVELS, lvl_blend, 0)
            pltpu.sync_copy(fb, out_ref.at[pl.ds(base, C)])
            return 0

        lax.fori_loop(0, chunks_per_w, chunk_body, 0)

    return k(x_hbm, y_hbm, table_hbm)


def _tc_decode(xy, z, feats, Wp0_t, Wp1_t, Wp2_t, Wmod_t, bmod, n):
    BN = 2048
    grid = (n // BN,)

    def body(xy_ref, z_ref, f_ref, wp0_ref, wp1_ref, wp2_ref, wm_ref, bm_ref,
             out_ref):
        x2 = xy_ref[...]
        encs = []
        for j in range(3):
            a = x2 * (2.0 ** j) * np.pi
            encs.append(jnp.sin(a))
            encs.append(jnp.cos(a))
        e = jnp.concatenate(encs, axis=-1)
        h = jax.nn.sigmoid(jnp.dot(e, wp0_ref[...],
                                   preferred_element_type=jnp.float32))
        h = jax.nn.sigmoid(jnp.dot(h, wp1_ref[...],
                                   preferred_element_type=jnp.float32))
        prior = jnp.dot(h, wp2_ref[...], preferred_element_type=jnp.float32)
        dec = jnp.dot(f_ref[...], wm_ref[...],
                      preferred_element_type=jnp.float32) + bm_ref[...]
        cols = lax.broadcasted_iota(jnp.int32, (1, 65), 1)
        col0 = (cols == 0).astype(jnp.float32)
        out_ref[...] = dec + (z_ref[...] - prior) * col0

    return pl.pallas_call(
        body,
        grid=grid,
        in_specs=[
            pl.BlockSpec((BN, 2), lambda i: (i, 0)),
            pl.BlockSpec((BN, 1), lambda i: (i, 0)),
            pl.BlockSpec((BN, 32), lambda i: (i, 0)),
            pl.BlockSpec((12, 64), lambda i: (0, 0)),
            pl.BlockSpec((64, 64), lambda i: (0, 0)),
            pl.BlockSpec((64, 1), lambda i: (0, 0)),
            pl.BlockSpec((32, 65), lambda i: (0, 0)),
            pl.BlockSpec((1, 65), lambda i: (0, 0)),
        ],
        out_specs=pl.BlockSpec((BN, 65), lambda i: (i, 0)),
        out_shape=jax.ShapeDtypeStruct((n, 65), jnp.float32),
    )(xy, z, feats, Wp0_t, Wp1_t, Wp2_t, Wmod_t, bmod)


def kernel(inputs, hash_table, W_tiny, b_tiny, Wp0, Wp1, Wp2):
    n = inputs.shape[0]
    x = inputs[:, 0]
    y = inputs[:, 1]
    xy = inputs[:, :2]
    z = inputs[:, 2:]
    table2d = hash_table.reshape(N_LEVELS * T, F_PER_LEVEL)

    feats = _sc_hashgrid(x, y, table2d, n)

    Wmod = W_tiny.at[0].multiply(-1.0)
    bmod = b_tiny.at[0].multiply(-1.0)
    out = _tc_decode(xy, z, feats, Wp0.T, Wp1.T, Wp2.T, Wmod.T,
                     bmod[None, :], n)
    return out
